# baseline (speedup 1.0000x reference)
import functools

import jax
import jax.numpy as jnp
from jax import lax
from jax.experimental import pallas as pl
from jax.experimental.pallas import tpu as pltpu
from jax.experimental.pallas import tpu_sc as plsc

B, T, D = 16, 2048, 512
NQ, S = 100, 16
NP = B * NQ * S
LANES = 16
NC, NS = 2, 16
NW = NC * NS
PPW = NP // NW
CHUNK = 80
NCH = PPW // CHUNK
DV = D // LANES


def _tc_prep(spans_ref, idxf_ref, idxc_ref, wb_ref):
    spans = spans_ref[...]
    start = spans[:, 0:1] * (T - 1)
    end = spans[:, 1:2] * (T - 1)
    base = (lax.broadcasted_iota(jnp.int32, (B * NQ, S), 0) // NQ) * T

    steps = lax.broadcasted_iota(jnp.int32, (B * NQ, S), 1).astype(
        jnp.float32) * (1.0 / (S - 1))
    sp = start + steps * (end - start)
    idxf = jnp.clip(sp.astype(jnp.int32), 0, T - 2)
    idxf_ref[...] = idxf + base
    idxc_ref[...] = idxf + base + 1

    s_col = lax.broadcasted_iota(jnp.int32, (B * NQ, S * LANES), 1) // LANES
    steps_b = s_col.astype(jnp.float32) * (1.0 / (S - 1))
    sp_b = start + steps_b * (end - start)
    idxf_b = jnp.clip(sp_b.astype(jnp.int32), 0, T - 2)
    wb_ref[...] = sp_b - idxf_b.astype(jnp.float32)


def _sc_body(table, idxf, idxc, wb, out,
             idxf_v, idxc_v, wb_v, fbuf, cbuf, semf, semc):
    wid = lax.axis_index("s") * NC + lax.axis_index("c")
    pltpu.sync_copy(idxf.at[wid], idxf_v)
    pltpu.sync_copy(idxc.at[wid], idxc_v)
    pltpu.sync_copy(wb.at[wid], wb_v)
    base_out = wid * PPW

    def chunk_body(j, carry):
        cf = pltpu.async_copy(table.at[idxf_v.at[j]], fbuf, semf)
        cc = pltpu.async_copy(table.at[idxc_v.at[j]], cbuf, semc)
        cf.wait()
        cc.wait()

        def pt_body(p, c2):
            wv = wb_v[j, p, :]
            for d in range(DV):
                sl = pl.ds(d * LANES, LANES)
                f = fbuf[p, sl]
                c = cbuf[p, sl]
                fbuf[p, sl] = f + wv * (c - f)
            return c2

        lax.fori_loop(0, CHUNK, pt_body, 0)
        pltpu.sync_copy(fbuf, out.at[pl.ds(base_out + j * CHUNK, CHUNK)])
        return carry

    lax.fori_loop(0, NCH, chunk_body, 0)


_sc_call = functools.partial(
    pl.kernel,
    mesh=plsc.VectorSubcoreMesh(core_axis_name="c", subcore_axis_name="s"),
    out_type=jax.ShapeDtypeStruct((NP, D), jnp.float32),
    scratch_types=[
        pltpu.VMEM((NCH, CHUNK), jnp.int32),
        pltpu.VMEM((NCH, CHUNK), jnp.int32),
        pltpu.VMEM((NCH, CHUNK, LANES), jnp.float32),
        pltpu.VMEM((CHUNK, D), jnp.float32),
        pltpu.VMEM((CHUNK, D), jnp.float32),
        pltpu.SemaphoreType.DMA,
        pltpu.SemaphoreType.DMA,
    ],
    compiler_params=pltpu.CompilerParams(use_tc_tiling_on_sc=False),
)(_sc_body)


def kernel(video_features, spans):
    table = video_features.reshape(B * T, D)
    idxf, idxc, wb = pl.pallas_call(
        _tc_prep,
        out_shape=[
            jax.ShapeDtypeStruct((B * NQ, S), jnp.int32),
            jax.ShapeDtypeStruct((B * NQ, S), jnp.int32),
            jax.ShapeDtypeStruct((B * NQ, S * LANES), jnp.float32),
        ],
    )(spans.reshape(B * NQ, 2))
    out = _sc_call(
        table,
        idxf.reshape(NW, NCH, CHUNK),
        idxc.reshape(NW, NCH, CHUNK),
        wb.reshape(NW, NCH, CHUNK, LANES),
    )
    return out.reshape(B, NQ, S, D)


# speedup vs baseline: 1.7650x; 1.7650x over previous
import functools

import jax
import jax.numpy as jnp
from jax import lax
from jax.experimental import pallas as pl
from jax.experimental.pallas import tpu as pltpu
from jax.experimental.pallas import tpu_sc as plsc

B, T, D = 16, 2048, 512
NQ, S = 100, 16
NP = B * NQ * S
LANES = 16
NC, NS = 2, 16
NW = NC * NS
PPW = NP // NW
CHUNK = 80
NCH = PPW // CHUNK
DV = D // LANES


def _tc_prep(spans_ref, idxf_ref, idxc_ref, wb_ref):
    spans = spans_ref[...]
    start = spans[:, 0:1] * (T - 1)
    end = spans[:, 1:2] * (T - 1)
    base = (lax.broadcasted_iota(jnp.int32, (B * NQ, S), 0) // NQ) * T

    steps = lax.broadcasted_iota(jnp.int32, (B * NQ, S), 1).astype(
        jnp.float32) * (1.0 / (S - 1))
    sp = start + steps * (end - start)
    idxf = jnp.clip(sp.astype(jnp.int32), 0, T - 2)
    idxf_ref[...] = idxf + base
    idxc_ref[...] = idxf + base + 1

    s_col = lax.broadcasted_iota(jnp.int32, (B * NQ, S * LANES), 1) // LANES
    steps_b = s_col.astype(jnp.float32) * (1.0 / (S - 1))
    sp_b = start + steps_b * (end - start)
    idxf_b = jnp.clip(sp_b.astype(jnp.int32), 0, T - 2)
    wb_ref[...] = sp_b - idxf_b.astype(jnp.float32)


def _sc_body(table, idxf, idxc, wb, out,
             idxf_v, idxc_v, wb_v, fbuf, cbuf, semf, semc):
    wid = lax.axis_index("s") * NC + lax.axis_index("c")
    pltpu.sync_copy(idxf.at[wid], idxf_v)
    pltpu.sync_copy(idxc.at[wid], idxc_v)
    pltpu.sync_copy(wb.at[wid], wb_v)
    base_out = wid * PPW

    def chunk_body(j, carry):
        cf = pltpu.async_copy(table.at[idxf_v.at[j]], fbuf, semf)
        cc = pltpu.async_copy(table.at[idxc_v.at[j]], cbuf, semc)
        cf.wait()
        cc.wait()

        def pt_body(p, c2):
            pg = j * CHUNK + p
            wv = wb_v[pg // 8, pl.ds((pg % 8) * LANES, LANES)]
            for d in range(DV):
                sl = pl.ds(d * LANES, LANES)
                f = fbuf[p, sl]
                c = cbuf[p, sl]
                fbuf[p, sl] = f + wv * (c - f)
            return c2

        lax.fori_loop(0, CHUNK, pt_body, 0)
        pltpu.sync_copy(fbuf, out.at[pl.ds(base_out + j * CHUNK, CHUNK)])
        return carry

    lax.fori_loop(0, NCH, chunk_body, 0)


_sc_call = functools.partial(
    pl.kernel,
    mesh=plsc.VectorSubcoreMesh(core_axis_name="c", subcore_axis_name="s"),
    out_type=jax.ShapeDtypeStruct((NP, D), jnp.float32),
    scratch_types=[
        pltpu.VMEM((NCH, CHUNK), jnp.int32),
        pltpu.VMEM((NCH, CHUNK), jnp.int32),
        pltpu.VMEM((PPW // 8, 8 * LANES), jnp.float32),
        pltpu.VMEM((CHUNK, D), jnp.float32),
        pltpu.VMEM((CHUNK, D), jnp.float32),
        pltpu.SemaphoreType.DMA,
        pltpu.SemaphoreType.DMA,
    ],
)(_sc_body)


def kernel(video_features, spans):
    table = video_features.reshape(B * T, D)
    idxf, idxc, wb = pl.pallas_call(
        _tc_prep,
        out_shape=[
            jax.ShapeDtypeStruct((B * NQ, S), jnp.int32),
            jax.ShapeDtypeStruct((B * NQ, S), jnp.int32),
            jax.ShapeDtypeStruct((B * NQ, S * LANES), jnp.float32),
        ],
    )(spans.reshape(B * NQ, 2))
    out = _sc_call(
        table,
        idxf.reshape(NW, NCH, CHUNK),
        idxc.reshape(NW, NCH, CHUNK),
        wb.reshape(NW, PPW // 8, 8 * LANES),
    )
    return out.reshape(B, NQ, S, D)
